# Initial kernel scaffold; baseline (speedup 1.0000x reference)
import jax
import jax.numpy as jnp
from jax.experimental import pallas as pl


def kernel(z, pos, batch, emb, mlp_W1, mlp_b1, mlp_W2, mlp_b2, cf_lin1_W, cf_lin2_W, cf_lin2_b, lin_W, lin_b, out1_W, out1_b, out2_W, out2_b):
    raise NotImplementedError("write your pallas kernel here")


# speedup vs baseline: 49.8731x; 49.8731x over previous
import functools
from math import pi as PI

import jax
import jax.numpy as jnp
from jax.experimental import pallas as pl
from jax.experimental.pallas import tpu as pltpu

T = 64
RBLK = 256
MAX_FAST = 768


def _ssp(x):
    return jax.nn.softplus(x) - jnp.log(2.0)


def _edge_kernel(prs, pcs, vals, h_ref, posr_ref, posc_ref, br_ref, bc_ref,
                 wcf1_ref, w1_ref, b1_ref, w2_ref, b2_ref, out_ref,
                 *, cutoff, ng):
    t = out_ref.shape[0]
    nf = out_ref.shape[1]
    i = pl.program_id(0)
    prev = pcs[jnp.maximum(i - 1, 0)]
    first = (i == 0) | (pcs[i] != prev)

    @pl.when(first)
    def _():
        out_ref[...] = jnp.zeros_like(out_ref)

    @pl.when(vals[i] == 1)
    def _():
        d23 = jnp.zeros((t, t, 1), jnp.float32)
        for k in range(3):
            cr = posr_ref[0, :, k:k + 1].reshape(t, 1, 1)
            cc = posc_ref[0, :, k:k + 1].reshape(1, t, 1)
            diff = cr - cc
            d23 = d23 + diff * diff
        r3 = jnp.sqrt(d23)
        b_r = br_ref[0].reshape(t, 1, 1)
        b_c = bc_ref[0].reshape(1, t, 1)
        ri = prs[i] * t + jax.lax.broadcasted_iota(jnp.int32, (t, 1, 1), 0)
        ci = pcs[i] * t + jax.lax.broadcasted_iota(jnp.int32, (1, t, 1), 1)
        mask3 = ((d23 <= cutoff * cutoff) & (b_r == b_c) & (ri != ci))
        cos3 = 0.5 * (jnp.cos(r3 * (PI / cutoff)) + 1.0)
        scale = jnp.where(mask3, cos3, 0.0).reshape(t * t, 1)
        rr = r3.reshape(t * t, 1)
        offs = (jax.lax.broadcasted_iota(jnp.int32, (1, ng), 1)
                .astype(jnp.float32) * (cutoff / (ng - 1)))
        coeff = -0.5 / (cutoff / (ng - 1)) ** 2
        ea = jnp.exp(coeff * (rr - offs) ** 2)
        hid = _ssp(jnp.dot(ea, w1_ref[...],
                           preferred_element_type=jnp.float32) + b1_ref[...])
        wf = jnp.dot(hid, w2_ref[...],
                     preferred_element_type=jnp.float32) + b2_ref[...]
        wm = (wf * scale).reshape(t, t, nf)
        x = jnp.dot(h_ref[...], wcf1_ref[...],
                    preferred_element_type=jnp.float32)
        out_ref[...] += jnp.sum(x.reshape(t, 1, nf) * wm, axis=0)


def _post_kernel(h_ref, agg_ref, wcf2_ref, bcf2_ref, wlin_ref, blin_ref,
                 out_ref):
    x = jnp.dot(agg_ref[...], wcf2_ref[...],
                preferred_element_type=jnp.float32) + bcf2_ref[...]
    x = _ssp(x)
    x = jnp.dot(x, wlin_ref[...],
                preferred_element_type=jnp.float32) + blin_ref[...]
    out_ref[...] = h_ref[...] + x


def _readout_kernel(h_ref, b_ref, w1_ref, b1_ref, w2_ref, b2_ref, out_ref):
    s = pl.program_id(0)
    ngraphs = out_ref.shape[0]
    x = _ssp(jnp.dot(h_ref[...], w1_ref[...],
                     preferred_element_type=jnp.float32) + b1_ref[...])
    y = jnp.dot(x, w2_ref[...],
                preferred_element_type=jnp.float32) + b2_ref[...]
    b = b_ref[0, 0, :]
    g = jax.lax.broadcasted_iota(jnp.int32, (ngraphs, b.shape[0]), 0)
    eq = (b[None, :] == g).astype(jnp.float32)
    part = jnp.sum(eq * y[:, 0][None, :], axis=1, keepdims=True)

    @pl.when(s == 0)
    def _():
        out_ref[...] = jnp.zeros_like(out_ref)

    out_ref[...] += part


def _edge_call(n, nf, hid, ng, cutoff, m, prs, pcs, vals, h, pos3, batch3,
               wcf1, w1, b1, w2, b2):
    body = functools.partial(_edge_kernel, cutoff=cutoff, ng=ng)
    grid_spec = pltpu.PrefetchScalarGridSpec(
        num_scalar_prefetch=3,
        grid=(m,),
        in_specs=[
            pl.BlockSpec((T, hid), lambda i, prs, pcs, vals: (prs[i], 0)),
            pl.BlockSpec((1, T, 3), lambda i, prs, pcs, vals: (prs[i], 0, 0)),
            pl.BlockSpec((1, T, 3), lambda i, prs, pcs, vals: (pcs[i], 0, 0)),
            pl.BlockSpec((1, T, 1), lambda i, prs, pcs, vals: (prs[i], 0, 0)),
            pl.BlockSpec((1, T, 1), lambda i, prs, pcs, vals: (pcs[i], 0, 0)),
            pl.BlockSpec((hid, nf), lambda i, *_: (0, 0)),
            pl.BlockSpec((ng, nf), lambda i, *_: (0, 0)),
            pl.BlockSpec((1, nf), lambda i, *_: (0, 0)),
            pl.BlockSpec((nf, nf), lambda i, *_: (0, 0)),
            pl.BlockSpec((1, nf), lambda i, *_: (0, 0)),
        ],
        out_specs=pl.BlockSpec((T, nf), lambda i, prs, pcs, vals: (pcs[i], 0)),
    )
    return pl.pallas_call(
        body,
        grid_spec=grid_spec,
        out_shape=jax.ShapeDtypeStruct((n, nf), jnp.float32),
        compiler_params=pltpu.CompilerParams(
            dimension_semantics=("arbitrary",)),
    )(prs, pcs, vals, h, pos3, pos3, batch3, batch3, wcf1, w1, b1, w2, b2)


def kernel(z, pos, batch, emb, mlp_W1, mlp_b1, mlp_W2, mlp_b2, cf_lin1_W,
           cf_lin2_W, cf_lin2_b, lin_W, lin_b, out1_W, out1_b, out2_W,
           out2_b):
    n = pos.shape[0]
    hid = emb.shape[1]
    ni, ng, nf = mlp_W1.shape
    cutoff = 10.0
    num_graphs = 256

    batch = batch.astype(jnp.int32)
    pos = pos.astype(jnp.float32)
    h0 = emb[z]

    p = n // T
    bt = batch.reshape(p, T)
    mins = bt[:, 0]
    maxs = bt[:, -1]
    ov = (mins[:, None] <= maxs[None, :]) & (maxs[:, None] >= mins[None, :])
    act = ov.T.reshape(-1)
    order = jnp.argsort(~act, stable=True).astype(jnp.int32)
    count = jnp.sum(act.astype(jnp.int32))
    last = order[count - 1]

    def build(m):
        idx = order[:m]
        valid = (jnp.arange(m, dtype=jnp.int32) < count)
        idx = jnp.where(valid, idx, last)
        return ((idx % p).astype(jnp.int32), (idx // p).astype(jnp.int32),
                valid.astype(jnp.int32))

    m_fast = min(MAX_FAST, p * p)
    fast = build(m_fast)
    slow = build(p * p)

    pos3 = pos.reshape(p, T, 3)
    batch3 = batch.reshape(p, T, 1)
    b1r = mlp_b1.reshape(ni, 1, nf)
    b2r = mlp_b2.reshape(ni, 1, nf)
    bcf2r = cf_lin2_b.reshape(ni, 1, hid)
    blinr = lin_b.reshape(ni, 1, hid)

    def edge_fast(h, wcf1, w1, b1, w2, b2):
        prs, pcs, vals = fast
        return _edge_call(n, nf, hid, ng, cutoff, m_fast, prs, pcs, vals,
                          h, pos3, batch3, wcf1, w1, b1, w2, b2)

    def edge_slow(h, wcf1, w1, b1, w2, b2):
        prs, pcs, vals = slow
        return _edge_call(n, nf, hid, ng, cutoff, p * p, prs, pcs, vals,
                          h, pos3, batch3, wcf1, w1, b1, w2, b2)

    nrb = n // RBLK

    def step(h, ws):
        wcf1, w1, b1, w2, b2, wcf2, bcf2, wlin, blin = ws
        agg = jax.lax.cond(count <= m_fast, edge_fast, edge_slow,
                           h, wcf1, w1, b1, w2, b2)
        h2 = pl.pallas_call(
            _post_kernel,
            grid=(nrb,),
            in_specs=[
                pl.BlockSpec((RBLK, hid), lambda i: (i, 0)),
                pl.BlockSpec((RBLK, nf), lambda i: (i, 0)),
                pl.BlockSpec((nf, hid), lambda i: (0, 0)),
                pl.BlockSpec((1, hid), lambda i: (0, 0)),
                pl.BlockSpec((hid, hid), lambda i: (0, 0)),
                pl.BlockSpec((1, hid), lambda i: (0, 0)),
            ],
            out_specs=pl.BlockSpec((RBLK, hid), lambda i: (i, 0)),
            out_shape=jax.ShapeDtypeStruct((n, hid), jnp.float32),
        )(h, agg, wcf2, bcf2, wlin, blin)
        return h2, None

    h, _ = jax.lax.scan(
        step, h0,
        (cf_lin1_W, mlp_W1, b1r, mlp_W2, b2r, cf_lin2_W, bcf2r, lin_W, blinr))

    hhalf = out1_W.shape[1]
    batch3r = batch.reshape(nrb, 1, RBLK)
    out = pl.pallas_call(
        _readout_kernel,
        grid=(nrb,),
        in_specs=[
            pl.BlockSpec((RBLK, hid), lambda i: (i, 0)),
            pl.BlockSpec((1, 1, RBLK), lambda i: (i, 0, 0)),
            pl.BlockSpec((hid, hhalf), lambda i: (0, 0)),
            pl.BlockSpec((1, hhalf), lambda i: (0, 0)),
            pl.BlockSpec((hhalf, 1), lambda i: (0, 0)),
            pl.BlockSpec((1, 1), lambda i: (0, 0)),
        ],
        out_specs=pl.BlockSpec((num_graphs, 1), lambda i: (0, 0)),
        out_shape=jax.ShapeDtypeStruct((num_graphs, 1), jnp.float32),
        compiler_params=pltpu.CompilerParams(
            dimension_semantics=("arbitrary",)),
    )(h, batch3r, out1_W, out1_b.reshape(1, hhalf), out2_W,
      out2_b.reshape(1, 1))
    return out


# speedup vs baseline: 144.3272x; 2.8939x over previous
import functools
from math import pi as PI

import jax
import jax.numpy as jnp
from jax.experimental import pallas as pl
from jax.experimental.pallas import tpu as pltpu

T = 64
RBLK = 256
MAX_FAST = 768


def _ssp(x):
    return jax.nn.softplus(x) - jnp.log(2.0)


def _edge_kernel(prs, pcs, vals, h_ref, posr_ref, posc_ref, br_ref, bc_ref,
                 wcf1_ref, w1_ref, b1_ref, w2_ref, b2_ref, out_ref,
                 *, cutoff, ng):
    t = out_ref.shape[0]
    nf = out_ref.shape[1]
    i = pl.program_id(0)
    prev = pcs[jnp.maximum(i - 1, 0)]
    first = (i == 0) | (pcs[i] != prev)

    @pl.when(first)
    def _():
        out_ref[...] = jnp.zeros_like(out_ref)

    @pl.when(vals[i] == 1)
    def _():
        d2 = jnp.zeros((t, t), jnp.float32)
        for k in range(3):
            cr = posr_ref[0, :, k:k + 1]
            cc = posc_ref[0, :, k:k + 1].reshape(1, t)
            diff = cr - cc
            d2 = d2 + diff * diff
        r = jnp.sqrt(d2)
        b_r = br_ref[0]
        b_c = bc_ref[0].reshape(1, t)
        ri = prs[i] * t + jax.lax.broadcasted_iota(jnp.int32, (t, 1), 0)
        ci = pcs[i] * t + jax.lax.broadcasted_iota(jnp.int32, (1, t), 1)
        mask = ((d2 <= cutoff * cutoff) & (b_r == b_c) & (ri != ci))
        v = d2 * (1.0 / (cutoff * cutoff))
        cpoly = 0.0
        for ck in (-1.3878952462213771e-07, 4.3030695870329447e-06,
                   -1.0463810492484571e-04, 1.9295743094039231e-03,
                   -2.5806891390014061e-02, 2.3533063035889320e-01,
                   -1.3352627688545895e+00, 4.0587121264167685e+00,
                   -4.9348022005446790e+00, 1.0):
            cpoly = cpoly * v + ck
        cosw = 0.5 * (cpoly + 1.0)
        scalew = jnp.where(mask, cosw, 0.0)
        scale = jnp.swapaxes(scalew.reshape(t, 1, t), 1, 2).reshape(t * t, 1)
        rr = jnp.swapaxes(r.reshape(t, 1, t), 1, 2).reshape(t * t, 1)
        offs = (jax.lax.broadcasted_iota(jnp.int32, (1, ng), 1)
                .astype(jnp.float32) * (cutoff / (ng - 1)))
        coeff = -0.5 / (cutoff / (ng - 1)) ** 2
        ea = jnp.exp(coeff * (rr - offs) ** 2)
        hid = _ssp(jnp.dot(ea, w1_ref[...],
                           preferred_element_type=jnp.float32) + b1_ref[...])
        wf = jnp.dot(hid, w2_ref[...],
                     preferred_element_type=jnp.float32) + b2_ref[...]
        wm = (wf * scale).reshape(t, t, nf)
        x = jnp.dot(h_ref[...], wcf1_ref[...],
                    preferred_element_type=jnp.float32)
        out_ref[...] += jnp.sum(x.reshape(t, 1, nf) * wm, axis=0)


def _post_kernel(h_ref, agg_ref, wcf2_ref, bcf2_ref, wlin_ref, blin_ref,
                 out_ref):
    x = jnp.dot(agg_ref[...], wcf2_ref[...],
                preferred_element_type=jnp.float32) + bcf2_ref[...]
    x = _ssp(x)
    x = jnp.dot(x, wlin_ref[...],
                preferred_element_type=jnp.float32) + blin_ref[...]
    out_ref[...] = h_ref[...] + x


def _readout_kernel(h_ref, b_ref, w1_ref, b1_ref, w2_ref, b2_ref, out_ref):
    s = pl.program_id(0)
    ngraphs = out_ref.shape[0]
    x = _ssp(jnp.dot(h_ref[...], w1_ref[...],
                     preferred_element_type=jnp.float32) + b1_ref[...])
    y = jnp.dot(x, w2_ref[...],
                preferred_element_type=jnp.float32) + b2_ref[...]
    b = b_ref[0, 0, :]
    g = jax.lax.broadcasted_iota(jnp.int32, (ngraphs, b.shape[0]), 0)
    eq = (b[None, :] == g).astype(jnp.float32)
    part = jnp.sum(eq * y[:, 0][None, :], axis=1, keepdims=True)

    @pl.when(s == 0)
    def _():
        out_ref[...] = jnp.zeros_like(out_ref)

    out_ref[...] += part


def _edge_call(n, nf, hid, ng, cutoff, m, prs, pcs, vals, h, pos3, batch3,
               wcf1, w1, b1, w2, b2):
    body = functools.partial(_edge_kernel, cutoff=cutoff, ng=ng)
    grid_spec = pltpu.PrefetchScalarGridSpec(
        num_scalar_prefetch=3,
        grid=(m,),
        in_specs=[
            pl.BlockSpec((T, hid), lambda i, prs, pcs, vals: (prs[i], 0)),
            pl.BlockSpec((1, T, 3), lambda i, prs, pcs, vals: (prs[i], 0, 0)),
            pl.BlockSpec((1, T, 3), lambda i, prs, pcs, vals: (pcs[i], 0, 0)),
            pl.BlockSpec((1, T, 1), lambda i, prs, pcs, vals: (prs[i], 0, 0)),
            pl.BlockSpec((1, T, 1), lambda i, prs, pcs, vals: (pcs[i], 0, 0)),
            pl.BlockSpec((hid, nf), lambda i, *_: (0, 0)),
            pl.BlockSpec((ng, nf), lambda i, *_: (0, 0)),
            pl.BlockSpec((1, nf), lambda i, *_: (0, 0)),
            pl.BlockSpec((nf, nf), lambda i, *_: (0, 0)),
            pl.BlockSpec((1, nf), lambda i, *_: (0, 0)),
        ],
        out_specs=pl.BlockSpec((T, nf), lambda i, prs, pcs, vals: (pcs[i], 0)),
    )
    return pl.pallas_call(
        body,
        grid_spec=grid_spec,
        out_shape=jax.ShapeDtypeStruct((n, nf), jnp.float32),
        compiler_params=pltpu.CompilerParams(
            dimension_semantics=("arbitrary",)),
    )(prs, pcs, vals, h, pos3, pos3, batch3, batch3, wcf1, w1, b1, w2, b2)


def kernel(z, pos, batch, emb, mlp_W1, mlp_b1, mlp_W2, mlp_b2, cf_lin1_W,
           cf_lin2_W, cf_lin2_b, lin_W, lin_b, out1_W, out1_b, out2_W,
           out2_b):
    n = pos.shape[0]
    hid = emb.shape[1]
    ni, ng, nf = mlp_W1.shape
    cutoff = 10.0
    num_graphs = 256

    batch = batch.astype(jnp.int32)
    pos = pos.astype(jnp.float32)
    h0 = emb[z]

    p = n // T
    bt = batch.reshape(p, T)
    mins = bt[:, 0]
    maxs = bt[:, -1]
    ov = (mins[:, None] <= maxs[None, :]) & (maxs[:, None] >= mins[None, :])
    act = ov.T.reshape(-1)
    order = jnp.argsort(~act, stable=True).astype(jnp.int32)
    count = jnp.sum(act.astype(jnp.int32))
    last = order[count - 1]

    def build(m):
        idx = order[:m]
        valid = (jnp.arange(m, dtype=jnp.int32) < count)
        idx = jnp.where(valid, idx, last)
        return ((idx % p).astype(jnp.int32), (idx // p).astype(jnp.int32),
                valid.astype(jnp.int32))

    m_fast = min(MAX_FAST, p * p)
    fast = build(m_fast)
    slow = build(p * p)

    pos3 = pos.reshape(p, T, 3)
    batch3 = batch.reshape(p, T, 1)
    b1r = mlp_b1.reshape(ni, 1, nf)
    b2r = mlp_b2.reshape(ni, 1, nf)
    bcf2r = cf_lin2_b.reshape(ni, 1, hid)
    blinr = lin_b.reshape(ni, 1, hid)

    def edge_fast(h, wcf1, w1, b1, w2, b2):
        prs, pcs, vals = fast
        return _edge_call(n, nf, hid, ng, cutoff, m_fast, prs, pcs, vals,
                          h, pos3, batch3, wcf1, w1, b1, w2, b2)

    def edge_slow(h, wcf1, w1, b1, w2, b2):
        prs, pcs, vals = slow
        return _edge_call(n, nf, hid, ng, cutoff, p * p, prs, pcs, vals,
                          h, pos3, batch3, wcf1, w1, b1, w2, b2)

    nrb = n // RBLK

    def step(h, ws):
        wcf1, w1, b1, w2, b2, wcf2, bcf2, wlin, blin = ws
        agg = jax.lax.cond(count <= m_fast, edge_fast, edge_slow,
                           h, wcf1, w1, b1, w2, b2)
        h2 = pl.pallas_call(
            _post_kernel,
            grid=(nrb,),
            in_specs=[
                pl.BlockSpec((RBLK, hid), lambda i: (i, 0)),
                pl.BlockSpec((RBLK, nf), lambda i: (i, 0)),
                pl.BlockSpec((nf, hid), lambda i: (0, 0)),
                pl.BlockSpec((1, hid), lambda i: (0, 0)),
                pl.BlockSpec((hid, hid), lambda i: (0, 0)),
                pl.BlockSpec((1, hid), lambda i: (0, 0)),
            ],
            out_specs=pl.BlockSpec((RBLK, hid), lambda i: (i, 0)),
            out_shape=jax.ShapeDtypeStruct((n, hid), jnp.float32),
        )(h, agg, wcf2, bcf2, wlin, blin)
        return h2, None

    h, _ = jax.lax.scan(
        step, h0,
        (cf_lin1_W, mlp_W1, b1r, mlp_W2, b2r, cf_lin2_W, bcf2r, lin_W, blinr))

    hhalf = out1_W.shape[1]
    batch3r = batch.reshape(nrb, 1, RBLK)
    out = pl.pallas_call(
        _readout_kernel,
        grid=(nrb,),
        in_specs=[
            pl.BlockSpec((RBLK, hid), lambda i: (i, 0)),
            pl.BlockSpec((1, 1, RBLK), lambda i: (i, 0, 0)),
            pl.BlockSpec((hid, hhalf), lambda i: (0, 0)),
            pl.BlockSpec((1, hhalf), lambda i: (0, 0)),
            pl.BlockSpec((hhalf, 1), lambda i: (0, 0)),
            pl.BlockSpec((1, 1), lambda i: (0, 0)),
        ],
        out_specs=pl.BlockSpec((num_graphs, 1), lambda i: (0, 0)),
        out_shape=jax.ShapeDtypeStruct((num_graphs, 1), jnp.float32),
        compiler_params=pltpu.CompilerParams(
            dimension_semantics=("arbitrary",)),
    )(h, batch3r, out1_W, out1_b.reshape(1, hhalf), out2_W,
      out2_b.reshape(1, 1))
    return out


# speedup vs baseline: 146.4816x; 1.0149x over previous
import functools
from math import pi as PI

import jax
import jax.numpy as jnp
from jax.experimental import pallas as pl
from jax.experimental.pallas import tpu as pltpu

T = 64
RBLK = 256
MAX_FAST = 512


def _ssp(x):
    return jax.nn.softplus(x) - jnp.log(2.0)


def _edge_kernel(prs, pcs, vals, h_ref, posr_ref, posc_ref, br_ref, bc_ref,
                 wcf1_ref, w1_ref, b1_ref, w2_ref, b2_ref, out_ref,
                 *, cutoff, ng):
    t = out_ref.shape[0]
    nf = out_ref.shape[1]
    i = pl.program_id(0)
    prev = pcs[jnp.maximum(i - 1, 0)]
    first = (i == 0) | (pcs[i] != prev)

    @pl.when(first)
    def _():
        out_ref[...] = jnp.zeros_like(out_ref)

    @pl.when(vals[i] == 1)
    def _():
        d2 = jnp.zeros((t, t), jnp.float32)
        for k in range(3):
            cr = posr_ref[0, :, k:k + 1]
            cc = posc_ref[0, :, k:k + 1].reshape(1, t)
            diff = cr - cc
            d2 = d2 + diff * diff
        r = jnp.sqrt(d2)
        b_r = br_ref[0]
        b_c = bc_ref[0].reshape(1, t)
        ri = prs[i] * t + jax.lax.broadcasted_iota(jnp.int32, (t, 1), 0)
        ci = pcs[i] * t + jax.lax.broadcasted_iota(jnp.int32, (1, t), 1)
        mask = ((d2 <= cutoff * cutoff) & (b_r == b_c) & (ri != ci))
        v = d2 * (1.0 / (cutoff * cutoff))
        cpoly = 0.0
        for ck in (-1.3878952462213771e-07, 4.3030695870329447e-06,
                   -1.0463810492484571e-04, 1.9295743094039231e-03,
                   -2.5806891390014061e-02, 2.3533063035889320e-01,
                   -1.3352627688545895e+00, 4.0587121264167685e+00,
                   -4.9348022005446790e+00, 1.0):
            cpoly = cpoly * v + ck
        cosw = 0.5 * (cpoly + 1.0)
        scalew = jnp.where(mask, cosw, 0.0)
        scale = jnp.swapaxes(scalew.reshape(t, 1, t), 1, 2).reshape(t * t, 1)
        rr = jnp.swapaxes(r.reshape(t, 1, t), 1, 2).reshape(t * t, 1)
        offs = (jax.lax.broadcasted_iota(jnp.int32, (1, ng), 1)
                .astype(jnp.float32) * (cutoff / (ng - 1)))
        coeff = -0.5 / (cutoff / (ng - 1)) ** 2
        ea = jnp.exp(coeff * (rr - offs) ** 2)
        hid = _ssp(jnp.dot(ea, w1_ref[...],
                           preferred_element_type=jnp.float32) + b1_ref[...])
        wf = jnp.dot(hid, w2_ref[...],
                     preferred_element_type=jnp.float32) + b2_ref[...]
        wm = (wf * scale).reshape(t, t, nf)
        x = jnp.dot(h_ref[...], wcf1_ref[...],
                    preferred_element_type=jnp.float32)
        out_ref[...] += jnp.sum(x.reshape(t, 1, nf) * wm, axis=0)


def _post_kernel(h_ref, agg_ref, wcf2_ref, bcf2_ref, wlin_ref, blin_ref,
                 out_ref):
    x = jnp.dot(agg_ref[...], wcf2_ref[...],
                preferred_element_type=jnp.float32) + bcf2_ref[...]
    x = _ssp(x)
    x = jnp.dot(x, wlin_ref[...],
                preferred_element_type=jnp.float32) + blin_ref[...]
    out_ref[...] = h_ref[...] + x


def _readout_kernel(h_ref, b_ref, w1_ref, b1_ref, w2_ref, b2_ref, out_ref):
    s = pl.program_id(0)
    ngraphs = out_ref.shape[0]
    x = _ssp(jnp.dot(h_ref[...], w1_ref[...],
                     preferred_element_type=jnp.float32) + b1_ref[...])
    y = jnp.dot(x, w2_ref[...],
                preferred_element_type=jnp.float32) + b2_ref[...]
    b = b_ref[0, 0, :]
    g = jax.lax.broadcasted_iota(jnp.int32, (ngraphs, b.shape[0]), 0)
    eq = (b[None, :] == g).astype(jnp.float32)
    part = jnp.sum(eq * y[:, 0][None, :], axis=1, keepdims=True)

    @pl.when(s == 0)
    def _():
        out_ref[...] = jnp.zeros_like(out_ref)

    out_ref[...] += part


def _edge_call(n, nf, hid, ng, cutoff, m, prs, pcs, vals, h, pos3, batch3,
               wcf1, w1, b1, w2, b2):
    body = functools.partial(_edge_kernel, cutoff=cutoff, ng=ng)
    grid_spec = pltpu.PrefetchScalarGridSpec(
        num_scalar_prefetch=3,
        grid=(m,),
        in_specs=[
            pl.BlockSpec((T, hid), lambda i, prs, pcs, vals: (prs[i], 0)),
            pl.BlockSpec((1, T, 3), lambda i, prs, pcs, vals: (prs[i], 0, 0)),
            pl.BlockSpec((1, T, 3), lambda i, prs, pcs, vals: (pcs[i], 0, 0)),
            pl.BlockSpec((1, T, 1), lambda i, prs, pcs, vals: (prs[i], 0, 0)),
            pl.BlockSpec((1, T, 1), lambda i, prs, pcs, vals: (pcs[i], 0, 0)),
            pl.BlockSpec((hid, nf), lambda i, *_: (0, 0)),
            pl.BlockSpec((ng, nf), lambda i, *_: (0, 0)),
            pl.BlockSpec((1, nf), lambda i, *_: (0, 0)),
            pl.BlockSpec((nf, nf), lambda i, *_: (0, 0)),
            pl.BlockSpec((1, nf), lambda i, *_: (0, 0)),
        ],
        out_specs=pl.BlockSpec((T, nf), lambda i, prs, pcs, vals: (pcs[i], 0)),
    )
    return pl.pallas_call(
        body,
        grid_spec=grid_spec,
        out_shape=jax.ShapeDtypeStruct((n, nf), jnp.float32),
        compiler_params=pltpu.CompilerParams(
            dimension_semantics=("arbitrary",)),
    )(prs, pcs, vals, h, pos3, pos3, batch3, batch3, wcf1, w1, b1, w2, b2)


def kernel(z, pos, batch, emb, mlp_W1, mlp_b1, mlp_W2, mlp_b2, cf_lin1_W,
           cf_lin2_W, cf_lin2_b, lin_W, lin_b, out1_W, out1_b, out2_W,
           out2_b):
    n = pos.shape[0]
    hid = emb.shape[1]
    ni, ng, nf = mlp_W1.shape
    cutoff = 10.0
    num_graphs = 256

    batch = batch.astype(jnp.int32)
    pos = pos.astype(jnp.float32)
    h0 = emb[z]

    p = n // T
    bt = batch.reshape(p, T)
    mins = bt[:, 0]
    maxs = bt[:, -1]
    ov = (mins[:, None] <= maxs[None, :]) & (maxs[:, None] >= mins[None, :])
    act = ov.T.reshape(-1)
    order = jnp.argsort(~act, stable=True).astype(jnp.int32)
    count = jnp.sum(act.astype(jnp.int32))
    last = order[count - 1]

    def build(m):
        idx = order[:m]
        valid = (jnp.arange(m, dtype=jnp.int32) < count)
        idx = jnp.where(valid, idx, last)
        return ((idx % p).astype(jnp.int32), (idx // p).astype(jnp.int32),
                valid.astype(jnp.int32))

    m_fast = min(MAX_FAST, p * p)
    fast = build(m_fast)
    slow = build(p * p)

    pos3 = pos.reshape(p, T, 3)
    batch3 = batch.reshape(p, T, 1)
    b1r = mlp_b1.reshape(ni, 1, nf)
    b2r = mlp_b2.reshape(ni, 1, nf)
    bcf2r = cf_lin2_b.reshape(ni, 1, hid)
    blinr = lin_b.reshape(ni, 1, hid)

    def edge_fast(h, wcf1, w1, b1, w2, b2):
        prs, pcs, vals = fast
        return _edge_call(n, nf, hid, ng, cutoff, m_fast, prs, pcs, vals,
                          h, pos3, batch3, wcf1, w1, b1, w2, b2)

    def edge_slow(h, wcf1, w1, b1, w2, b2):
        prs, pcs, vals = slow
        return _edge_call(n, nf, hid, ng, cutoff, p * p, prs, pcs, vals,
                          h, pos3, batch3, wcf1, w1, b1, w2, b2)

    nrb = n // RBLK

    def step(h, ws):
        wcf1, w1, b1, w2, b2, wcf2, bcf2, wlin, blin = ws
        agg = jax.lax.cond(count <= m_fast, edge_fast, edge_slow,
                           h, wcf1, w1, b1, w2, b2)
        h2 = pl.pallas_call(
            _post_kernel,
            grid=(nrb,),
            in_specs=[
                pl.BlockSpec((RBLK, hid), lambda i: (i, 0)),
                pl.BlockSpec((RBLK, nf), lambda i: (i, 0)),
                pl.BlockSpec((nf, hid), lambda i: (0, 0)),
                pl.BlockSpec((1, hid), lambda i: (0, 0)),
                pl.BlockSpec((hid, hid), lambda i: (0, 0)),
                pl.BlockSpec((1, hid), lambda i: (0, 0)),
            ],
            out_specs=pl.BlockSpec((RBLK, hid), lambda i: (i, 0)),
            out_shape=jax.ShapeDtypeStruct((n, hid), jnp.float32),
        )(h, agg, wcf2, bcf2, wlin, blin)
        return h2, None

    h, _ = jax.lax.scan(
        step, h0,
        (cf_lin1_W, mlp_W1, b1r, mlp_W2, b2r, cf_lin2_W, bcf2r, lin_W, blinr))

    hhalf = out1_W.shape[1]
    batch3r = batch.reshape(nrb, 1, RBLK)
    out = pl.pallas_call(
        _readout_kernel,
        grid=(nrb,),
        in_specs=[
            pl.BlockSpec((RBLK, hid), lambda i: (i, 0)),
            pl.BlockSpec((1, 1, RBLK), lambda i: (i, 0, 0)),
            pl.BlockSpec((hid, hhalf), lambda i: (0, 0)),
            pl.BlockSpec((1, hhalf), lambda i: (0, 0)),
            pl.BlockSpec((hhalf, 1), lambda i: (0, 0)),
            pl.BlockSpec((1, 1), lambda i: (0, 0)),
        ],
        out_specs=pl.BlockSpec((num_graphs, 1), lambda i: (0, 0)),
        out_shape=jax.ShapeDtypeStruct((num_graphs, 1), jnp.float32),
        compiler_params=pltpu.CompilerParams(
            dimension_semantics=("arbitrary",)),
    )(h, batch3r, out1_W, out1_b.reshape(1, hhalf), out2_W,
      out2_b.reshape(1, 1))
    return out
